# baseline (speedup 1.0000x reference)
import functools

import jax
import jax.numpy as jnp
from jax import lax
from jax.experimental import pallas as pl
from jax.experimental.pallas import tpu as pltpu
from jax.experimental.pallas import tpu_sc as plsc

N = 10000
D = 128
E = 320000
NC = 2
NS = 16
NW = NC * NS
EPW = E // NW
W = 80
CHUNKS = EPW // W
PAIRS = (CHUNKS - 1) // 2
NG = W // 16
SSTRIDE = 17


def _make_kernel():
    f32 = jnp.float32
    i32 = jnp.int32
    out_sd = jax.ShapeDtypeStruct((E,), f32)
    mesh = plsc.VectorSubcoreMesh(core_axis_name="c", subcore_axis_name="s")

    @functools.partial(
        pl.kernel,
        out_type=(out_sd, out_sd),
        mesh=mesh,
        compiler_params=pltpu.CompilerParams(needs_layout_passes=False),
        scratch_types=[
            pltpu.VMEM_SHARED((N, D), f32),
            pltpu.VMEM((W,), i32),
            pltpu.VMEM((W,), i32),
            pltpu.VMEM((W,), i32),
            pltpu.VMEM((W,), i32),
            pltpu.VMEM((W, D), f32),
            pltpu.VMEM((W, D), f32),
            pltpu.VMEM((W, D), f32),
            pltpu.VMEM((W, D), f32),
            pltpu.VMEM((W,), f32),
            pltpu.VMEM((W,), f32),
            pltpu.VMEM((W * SSTRIDE,), f32),
            pltpu.SemaphoreType.DMA,
            pltpu.SemaphoreType.DMA,
            pltpu.SemaphoreType.DMA,
            pltpu.SemaphoreType.DMA,
            pltpu.SemaphoreType.DMA,
            pltpu.SemaphoreType.DMA,
            pltpu.SemaphoreType.DMA,
            pltpu.SemaphoreType.DMA,
        ],
    )
    def k(pqi, pai, nqi, nai, xq, xa, pos_out, neg_out,
          sq, iq0, ia0, iq1, ia1, qr0, ar0, qr1, ar1, ov0, ov1, scr,
          is0, is1, qs0, as0, qs1, as1, os0, os1):
        cid = lax.axis_index("c")
        sid = lax.axis_index("s")
        wid = sid * NC + cid
        col = lax.iota(jnp.int32, 16) * SSTRIDE

        @pl.when(sid == 0)
        def _():
            pltpu.sync_copy(xq, sq)

        plsc.subcore_barrier()

        def do_set(idxq_hbm, idxa_hbm, out_hbm):
            base = wid * EPW

            def fetch_idx(c, iq, ia, isem):
                pltpu.async_copy(idxq_hbm.at[pl.ds(base + c * W, W)], iq, isem)
                pltpu.async_copy(idxa_hbm.at[pl.ds(base + c * W, W)], ia, isem)

            def wait_idx(c, iq, ia, isem):
                pltpu.make_async_copy(
                    idxq_hbm.at[pl.ds(base + c * W, W)], iq, isem).wait()
                pltpu.make_async_copy(
                    idxa_hbm.at[pl.ds(base + c * W, W)], ia, isem).wait()

            def start_gather(iq, ia, qr, ar, qsem, asem):
                pltpu.async_copy(sq.at[iq], qr, qsem)
                pltpu.async_copy(xa.at[ia], ar, asem)

            def wait_gather(iq, ia, qr, ar, qsem, asem):
                pltpu.make_async_copy(sq.at[iq], qr, qsem).wait()
                pltpu.make_async_copy(xa.at[ia], ar, asem).wait()

            def compute(qr, ar, ov):
                @plsc.parallel_loop(0, W, unroll=4)
                def _(e):
                    acc = qr[e, pl.ds(0, 16)] * ar[e, pl.ds(0, 16)]
                    for j in range(1, D // 16):
                        acc = acc + (qr[e, pl.ds(16 * j, 16)]
                                     * ar[e, pl.ds(16 * j, 16)])
                    scr[pl.ds(e * SSTRIDE, 16)] = acc

                @plsc.parallel_loop(0, NG)
                def _(g):
                    colg = col + g * (16 * SSTRIDE)
                    dot = plsc.load_gather(scr, [colg])
                    for l in range(1, 16):
                        dot = dot + plsc.load_gather(scr, [colg + l])
                    ov[pl.ds(g * 16, 16)] = 1.0 / (1.0 + jnp.exp(-dot))

            def put_out(c, ov, osem):
                pltpu.async_copy(ov, out_hbm.at[pl.ds(base + c * W, W)], osem)

            def wait_out(c, ov, osem):
                pltpu.make_async_copy(
                    ov, out_hbm.at[pl.ds(base + c * W, W)], osem).wait()

            fetch_idx(0, iq0, ia0, is0)
            fetch_idx(1, iq1, ia1, is1)
            wait_idx(0, iq0, ia0, is0)
            start_gather(iq0, ia0, qr0, ar0, qs0, as0)
            wait_idx(1, iq1, ia1, is1)
            start_gather(iq1, ia1, qr1, ar1, qs1, as1)

            @pl.loop(0, PAIRS)
            def _(c2):
                c = 2 * c2

                wait_gather(iq0, ia0, qr0, ar0, qs0, as0)
                fetch_idx(c + 2, iq0, ia0, is0)

                @pl.when(c2 > 0)
                def _():
                    wait_out(c - 2, ov0, os0)

                compute(qr0, ar0, ov0)
                put_out(c, ov0, os0)
                wait_idx(c + 2, iq0, ia0, is0)
                start_gather(iq0, ia0, qr0, ar0, qs0, as0)

                wait_gather(iq1, ia1, qr1, ar1, qs1, as1)

                @pl.when(c2 < PAIRS - 1)
                def _():
                    fetch_idx(c + 3, iq1, ia1, is1)

                @pl.when(c2 > 0)
                def _():
                    wait_out(c - 1, ov1, os1)

                compute(qr1, ar1, ov1)
                put_out(c + 1, ov1, os1)

                @pl.when(c2 < PAIRS - 1)
                def _():
                    wait_idx(c + 3, iq1, ia1, is1)
                    start_gather(iq1, ia1, qr1, ar1, qs1, as1)

            wait_gather(iq0, ia0, qr0, ar0, qs0, as0)
            wait_out(CHUNKS - 3, ov0, os0)
            compute(qr0, ar0, ov0)
            put_out(CHUNKS - 1, ov0, os0)
            wait_out(CHUNKS - 2, ov1, os1)
            wait_out(CHUNKS - 1, ov0, os0)

        do_set(pqi, pai, pos_out)
        do_set(nqi, nai, neg_out)

    return k


_edge_decoder = _make_kernel()


@jax.jit
def kernel(x_question, x_answer, pos_edge_label_index, neg_edge_label_index):
    return _edge_decoder(
        pos_edge_label_index[0], pos_edge_label_index[1],
        neg_edge_label_index[0], neg_edge_label_index[1],
        x_question, x_answer)


# speedup vs baseline: 1.0095x; 1.0095x over previous
import functools

import jax
import jax.numpy as jnp
from jax import lax
from jax.experimental import pallas as pl
from jax.experimental.pallas import tpu as pltpu
from jax.experimental.pallas import tpu_sc as plsc

N = 10000
D = 128
E = 320000
NC = 2
NS = 16
NW = NC * NS
EPW = E // NW
W = 80
CHUNKS = EPW // W
PAIRS = (CHUNKS - 1) // 2
NG = W // 16
SSTRIDE = 17


def _make_kernel():
    f32 = jnp.float32
    i32 = jnp.int32
    out_sd = jax.ShapeDtypeStruct((E,), f32)
    mesh = plsc.VectorSubcoreMesh(core_axis_name="c", subcore_axis_name="s")

    @functools.partial(
        pl.kernel,
        out_type=(out_sd, out_sd),
        mesh=mesh,
        compiler_params=pltpu.CompilerParams(needs_layout_passes=False),
        scratch_types=[
            pltpu.VMEM_SHARED((N, D), f32),
            pltpu.VMEM((W,), i32),
            pltpu.VMEM((W,), i32),
            pltpu.VMEM((W,), i32),
            pltpu.VMEM((W,), i32),
            pltpu.VMEM((W,), i32),
            pltpu.VMEM((W,), i32),
            pltpu.VMEM((W,), i32),
            pltpu.VMEM((W,), i32),
            pltpu.VMEM((W, D), f32),
            pltpu.VMEM((W, D), f32),
            pltpu.VMEM((W, D), f32),
            pltpu.VMEM((W, D), f32),
            pltpu.VMEM((W,), f32),
            pltpu.VMEM((W,), f32),
            pltpu.VMEM((W * SSTRIDE,), f32),
            pltpu.SemaphoreType.DMA,
            pltpu.SemaphoreType.DMA,
            pltpu.SemaphoreType.DMA,
            pltpu.SemaphoreType.DMA,
            pltpu.SemaphoreType.DMA,
            pltpu.SemaphoreType.DMA,
            pltpu.SemaphoreType.DMA,
            pltpu.SemaphoreType.DMA,
            pltpu.SemaphoreType.DMA,
            pltpu.SemaphoreType.DMA,
        ],
    )
    def k(pqi, pai, nqi, nai, xq, xa, pos_out, neg_out,
          sq, iq0, ia0, iq1, ia1, iq2, ia2, iq3, ia3,
          qr0, ar0, qr1, ar1, ov0, ov1, scr,
          is0, is1, is2, is3, qs0, as0, qs1, as1, os0, os1):
        cid = lax.axis_index("c")
        sid = lax.axis_index("s")
        wid = sid * NC + cid
        col = lax.iota(jnp.int32, 16) * SSTRIDE

        @pl.when(sid == 0)
        def _():
            pltpu.sync_copy(xq, sq)

        plsc.subcore_barrier()

        def do_set(idxq_hbm, idxa_hbm, out_hbm):
            base = wid * EPW

            def fetch_idx(c, iq, ia, isem):
                pltpu.async_copy(idxq_hbm.at[pl.ds(base + c * W, W)], iq, isem)
                pltpu.async_copy(idxa_hbm.at[pl.ds(base + c * W, W)], ia, isem)

            def wait_idx(c, iq, ia, isem):
                pltpu.make_async_copy(
                    idxq_hbm.at[pl.ds(base + c * W, W)], iq, isem).wait()
                pltpu.make_async_copy(
                    idxa_hbm.at[pl.ds(base + c * W, W)], ia, isem).wait()

            def start_gather(iq, ia, qr, ar, qsem, asem):
                pltpu.async_copy(sq.at[iq], qr, qsem)
                pltpu.async_copy(xa.at[ia], ar, asem)

            def wait_gather(iq, ia, qr, ar, qsem, asem):
                pltpu.make_async_copy(sq.at[iq], qr, qsem).wait()
                pltpu.make_async_copy(xa.at[ia], ar, asem).wait()

            def compute(qr, ar, ov):
                @plsc.parallel_loop(0, W, unroll=2)
                def _(e):
                    acc = qr[e, pl.ds(0, 16)] * ar[e, pl.ds(0, 16)]
                    for j in range(1, D // 16):
                        acc = acc + (qr[e, pl.ds(16 * j, 16)]
                                     * ar[e, pl.ds(16 * j, 16)])
                    scr[pl.ds(e * SSTRIDE, 16)] = acc

                @plsc.parallel_loop(0, NG)
                def _(g):
                    colg = col + g * (16 * SSTRIDE)
                    dot = plsc.load_gather(scr, [colg])
                    for l in range(1, 16):
                        dot = dot + plsc.load_gather(scr, [colg + l])
                    ov[pl.ds(g * 16, 16)] = 1.0 / (1.0 + jnp.exp(-dot))

            def put_out(c, ov, osem):
                pltpu.async_copy(ov, out_hbm.at[pl.ds(base + c * W, W)], osem)

            def wait_out(c, ov, osem):
                pltpu.make_async_copy(
                    ov, out_hbm.at[pl.ds(base + c * W, W)], osem).wait()

            islots = ((iq0, ia0, is0), (iq1, ia1, is1),
                      (iq2, ia2, is2), (iq3, ia3, is3))
            rslots = ((qr0, ar0, qs0, as0, ov0, os0),
                      (qr1, ar1, qs1, as1, ov1, os1))

            for s in range(4):
                fetch_idx(s, *islots[s])
            for s in range(2):
                wait_idx(s, *islots[s])
                start_gather(islots[s][0], islots[s][1], *rslots[s][:4])

            NQUAD = (CHUNKS - 1) // 4

            @pl.loop(0, NQUAD)
            def _(c3):
                c = 4 * c3

                for p in range(4):
                    ch = c + p
                    qr, ar, qsem, asem, ov, osem = rslots[p % 2]
                    iqf, iaf, isf = islots[p]
                    iqg, iag, isg = islots[(p + 2) % 4]

                    wait_gather(islots[p][0], islots[p][1],
                                qr, ar, qsem, asem)

                    if p == 0:
                        fetch_idx(ch + 4, iqf, iaf, isf)
                    else:
                        @pl.when(c3 < NQUAD - 1)
                        def _():
                            fetch_idx(ch + 4, iqf, iaf, isf)

                    if p < 2:
                        @pl.when(c3 > 0)
                        def _():
                            wait_out(ch - 2, ov, osem)
                    else:
                        wait_out(ch - 2, ov, osem)

                    compute(qr, ar, ov)
                    put_out(ch, ov, osem)

                    if p == 3:
                        @pl.when(c3 < NQUAD - 1)
                        def _():
                            wait_idx(ch + 2, iqg, iag, isg)
                            start_gather(iqg, iag, qr, ar, qsem, asem)
                    else:
                        wait_idx(ch + 2, iqg, iag, isg)
                        start_gather(iqg, iag, qr, ar, qsem, asem)

            wait_gather(iq0, ia0, qr0, ar0, qs0, as0)
            wait_out(CHUNKS - 3, ov0, os0)
            compute(qr0, ar0, ov0)
            put_out(CHUNKS - 1, ov0, os0)
            wait_out(CHUNKS - 2, ov1, os1)
            wait_out(CHUNKS - 1, ov0, os0)

        do_set(pqi, pai, pos_out)
        do_set(nqi, nai, neg_out)

    return k


_edge_decoder = _make_kernel()


@jax.jit
def kernel(x_question, x_answer, pos_edge_label_index, neg_edge_label_index):
    return _edge_decoder(
        pos_edge_label_index[0], pos_edge_label_index[1],
        neg_edge_label_index[0], neg_edge_label_index[1],
        x_question, x_answer)
